# baseline (speedup 1.0000x reference)
import jax
import jax.numpy as jnp
from jax.experimental import pallas as pl

_EPS = 1e-8
_K = 64
_V = 32768
_R = 32
_ROWS = 512
_NCH = 32


def _keys(x):
    b = jax.lax.bitcast_convert_type(x, jnp.int32)
    return jnp.where(b >= 0, b, b ^ jnp.int32(0x7FFFFFFF))


def _count_ge16(data, cand):
    chw = _V // _NCH
    acc = jnp.zeros((data.shape[0], chw), jnp.int16)
    for c in range(_NCH):
        blk = jax.lax.slice_in_dim(data, c * chw, (c + 1) * chw, axis=1)
        acc = acc + jnp.where(blk >= cand, jnp.int16(1), jnp.int16(0))
    return jnp.sum(acc.astype(jnp.int32), axis=-1, keepdims=True)


def _kth16(data, k):
    cnt0 = _count_ge16(data, jnp.zeros((data.shape[0], 1), jnp.int16))
    prefix = jnp.where(cnt0 >= k, jnp.int32(0), jnp.int32(-32768))

    def body(i, prefix):
        bit = jnp.left_shift(jnp.int32(1), 14 - i)
        cand = prefix | bit
        cnt = _count_ge16(data, cand.astype(jnp.int16))
        return jnp.where(cnt >= k, cand, prefix)

    return jax.lax.fori_loop(0, 15, body, prefix)


def _halves(x):
    key = _keys(x)
    hi = jnp.right_shift(key, 16).astype(jnp.int16)
    lo = ((key & jnp.int32(0xFFFF)) - jnp.int32(32768)).astype(jnp.int16)
    return hi, lo


def _kth_largest_key(hi, lo, k):
    rows = hi.shape[0]
    kvec = jnp.full((rows, 1), k, jnp.int32)
    h32 = _kth16(hi, kvec)

    c_top = jnp.where(
        h32 >= 32767,
        jnp.zeros_like(kvec),
        _count_ge16(hi, (h32 + 1).astype(jnp.int16)),
    )
    k2 = kvec - c_top

    sentinel = jnp.int16(-32768)
    lo_masked = jnp.where(hi == h32.astype(jnp.int16), lo, sentinel)
    lo_thr = _kth16(lo_masked, k2)

    lo_u = (lo_thr + jnp.int32(32768)) & jnp.int32(0xFFFF)
    return jnp.left_shift(h32, 16) | lo_u


def _kl_body(p_ref, q_ref, o_ref):
    p = p_ref[...]
    q = q_ref[...]
    hp, lp = _halves(p)
    hq, lq = _halves(q)
    thr = _kth_largest_key(
        jnp.concatenate([hp, hq], axis=0),
        jnp.concatenate([lp, lq], axis=0),
        _K,
    )
    thrf = jax.lax.bitcast_convert_type(
        jnp.where(thr >= 0, thr, thr ^ jnp.int32(0x7FFFFFFF)), jnp.float32
    )
    mask = (p >= thrf[:_R]) | (q >= thrf[_R:])

    mp = jnp.max(p, axis=-1, keepdims=True)
    mq = jnp.max(q, axis=-1, keepdims=True)
    ep = jnp.where(mask, jnp.exp(p - mp), 0.0)
    eq = jnp.where(mask, jnp.exp(q - mq), 0.0)
    sp = jnp.sum(ep, axis=-1, keepdims=True)
    sq = jnp.sum(eq, axis=-1, keepdims=True)
    pn = ep * (1.0 / sp) + _EPS
    qn = eq * (1.0 / sq) + _EPS
    t = (pn - qn) * (jnp.log(pn) - jnp.log(qn))
    z = jnp.float32(1.0 + _V * _EPS)
    o_ref[...] = (0.5 / z) * jnp.sum(t, axis=-1, keepdims=True)


@jax.jit
def kernel(logits_p, logits_q):
    p = logits_p.reshape(_ROWS, _V)
    q = logits_q.reshape(_ROWS, _V)
    out = pl.pallas_call(
        _kl_body,
        grid=(_ROWS // _R,),
        in_specs=[
            pl.BlockSpec((_R, _V), lambda i: (i, 0)),
            pl.BlockSpec((_R, _V), lambda i: (i, 0)),
        ],
        out_specs=pl.BlockSpec((_R, 1), lambda i: (i, 0)),
        out_shape=jax.ShapeDtypeStruct((_ROWS, 1), jnp.float32),
    )(p, q)
    return out.reshape(logits_p.shape[0], logits_p.shape[1])


# speedup vs baseline: 1.1190x; 1.1190x over previous
import jax
import jax.numpy as jnp
from jax.experimental import pallas as pl

_EPS = 1e-8
_K = 64
_V = 32768
_R = 32
_ROWS = 512
_NCH = 32


def _keys(x):
    b = jax.lax.bitcast_convert_type(x, jnp.int32)
    return jnp.where(b >= 0, b, b ^ jnp.int32(0x7FFFFFFF))


def _count_ge16(data, cand):
    chw = _V // _NCH
    acc = jnp.zeros((data.shape[0], chw), jnp.int16)
    for c in range(_NCH):
        blk = jax.lax.slice_in_dim(data, c * chw, (c + 1) * chw, axis=1)
        acc = jnp.where(blk >= cand, acc + jnp.int16(1), acc)
    return jnp.sum(acc.astype(jnp.int32), axis=-1, keepdims=True)


def _kth16(data, k, done):
    cnt0 = _count_ge16(data, jnp.zeros((data.shape[0], 1), jnp.int16))
    prefix = jnp.where(cnt0 >= k, jnp.int32(0), jnp.int32(-32768))
    done = done | (cnt0 == k).astype(jnp.int32)

    def cond(state):
        i, _, done = state
        return jnp.logical_and(i < 15, jnp.logical_not(jnp.all(done > 0)))

    def body(state):
        i, prefix, done = state
        bit = jnp.left_shift(jnp.int32(1), 14 - i)
        cand = prefix | bit
        cnt = _count_ge16(data, cand.astype(jnp.int16))
        live_take = jnp.logical_and(cnt >= k, done == 0)
        prefix = jnp.where(live_take, cand, prefix)
        done = done | (cnt == k).astype(jnp.int32)
        return i + 1, prefix, done

    _, prefix, done = jax.lax.while_loop(cond, body, (jnp.int32(0), prefix, done))
    return prefix, done


def _halves(x):
    key = _keys(x)
    hi = jnp.right_shift(key, 16).astype(jnp.int16)
    lo = ((key & jnp.int32(0xFFFF)) - jnp.int32(32768)).astype(jnp.int16)
    return hi, lo


def _kth_largest_key(hi, lo, k):
    rows = hi.shape[0]
    kvec = jnp.full((rows, 1), k, jnp.int32)
    h32, done1 = _kth16(hi, kvec, jnp.zeros((rows, 1), jnp.int32))

    c_top = jnp.where(
        h32 >= 32767,
        jnp.zeros_like(kvec),
        _count_ge16(hi, (h32 + 1).astype(jnp.int16)),
    )
    k2 = jnp.maximum(kvec - c_top, 1)

    sentinel = jnp.int16(-32768)
    lo_masked = jnp.where(hi == h32.astype(jnp.int16), lo, sentinel)
    lo_thr, _ = _kth16(lo_masked, k2, done1)

    lo_u = (lo_thr + jnp.int32(32768)) & jnp.int32(0xFFFF)
    full = jnp.left_shift(h32, 16) | lo_u
    return jnp.where(done1 > 0, jnp.left_shift(h32, 16), full)


def _kl_body(p_ref, q_ref, o_ref):
    p = p_ref[...]
    q = q_ref[...]
    hp, lp = _halves(p)
    hq, lq = _halves(q)
    thr = _kth_largest_key(
        jnp.concatenate([hp, hq], axis=0),
        jnp.concatenate([lp, lq], axis=0),
        _K,
    )
    thrf = jax.lax.bitcast_convert_type(
        jnp.where(thr >= 0, thr, thr ^ jnp.int32(0x7FFFFFFF)), jnp.float32
    )
    mask = (p >= thrf[:_R]) | (q >= thrf[_R:])

    mp = jnp.max(p, axis=-1, keepdims=True)
    mq = jnp.max(q, axis=-1, keepdims=True)
    ep = jnp.where(mask, jnp.exp(p - mp), 0.0)
    eq = jnp.where(mask, jnp.exp(q - mq), 0.0)
    sp = jnp.sum(ep, axis=-1, keepdims=True)
    sq = jnp.sum(eq, axis=-1, keepdims=True)
    pn = ep * (1.0 / sp) + _EPS
    qn = eq * (1.0 / sq) + _EPS
    t = (pn - qn) * (jnp.log(pn) - jnp.log(qn))
    z = jnp.float32(1.0 + _V * _EPS)
    o_ref[...] = (0.5 / z) * jnp.sum(t, axis=-1, keepdims=True)


@jax.jit
def kernel(logits_p, logits_q):
    p = logits_p.reshape(_ROWS, _V)
    q = logits_q.reshape(_ROWS, _V)
    out = pl.pallas_call(
        _kl_body,
        grid=(_ROWS // _R,),
        in_specs=[
            pl.BlockSpec((_R, _V), lambda i: (i, 0)),
            pl.BlockSpec((_R, _V), lambda i: (i, 0)),
        ],
        out_specs=pl.BlockSpec((_R, 1), lambda i: (i, 0)),
        out_shape=jax.ShapeDtypeStruct((_ROWS, 1), jnp.float32),
    )(p, q)
    return out.reshape(logits_p.shape[0], logits_p.shape[1])


# speedup vs baseline: 1.1246x; 1.0050x over previous
import jax
import jax.numpy as jnp
from jax.experimental import pallas as pl

_EPS = 1e-8
_K = 64
_V = 32768
_R = 32
_ROWS = 512
_NCH = 64


def _keys(x):
    b = jax.lax.bitcast_convert_type(x, jnp.int32)
    return jnp.where(b >= 0, b, b ^ jnp.int32(0x7FFFFFFF))


def _count_ge16(data, cand):
    chw = _V // _NCH
    acc = jnp.zeros((data.shape[0], chw), jnp.int16)
    for c in range(_NCH):
        blk = jax.lax.slice_in_dim(data, c * chw, (c + 1) * chw, axis=1)
        acc = jnp.where(blk >= cand, acc + jnp.int16(1), acc)
    return jnp.sum(acc.astype(jnp.int32), axis=-1, keepdims=True)


def _kth16(data, k, done):
    cnt0 = _count_ge16(data, jnp.zeros((data.shape[0], 1), jnp.int16))
    prefix = jnp.where(cnt0 >= k, jnp.int32(0), jnp.int32(-32768))
    done = done | (cnt0 == k).astype(jnp.int32)

    def cond(state):
        i, _, done = state
        return jnp.logical_and(i < 15, jnp.logical_not(jnp.all(done > 0)))

    def body(state):
        i, prefix, done = state
        bit = jnp.left_shift(jnp.int32(1), 14 - i)
        cand = prefix | bit
        cnt = _count_ge16(data, cand.astype(jnp.int16))
        live_take = jnp.logical_and(cnt >= k, done == 0)
        prefix = jnp.where(live_take, cand, prefix)
        done = done | (cnt == k).astype(jnp.int32)
        return i + 1, prefix, done

    _, prefix, done = jax.lax.while_loop(cond, body, (jnp.int32(0), prefix, done))
    return prefix, done


def _halves(x):
    key = _keys(x)
    hi = jnp.right_shift(key, 16).astype(jnp.int16)
    lo = ((key & jnp.int32(0xFFFF)) - jnp.int32(32768)).astype(jnp.int16)
    return hi, lo


def _kth_largest_key(hi, lo, k):
    rows = hi.shape[0]
    kvec = jnp.full((rows, 1), k, jnp.int32)
    h32, done1 = _kth16(hi, kvec, jnp.zeros((rows, 1), jnp.int32))

    c_top = jnp.where(
        h32 >= 32767,
        jnp.zeros_like(kvec),
        _count_ge16(hi, (h32 + 1).astype(jnp.int16)),
    )
    k2 = jnp.maximum(kvec - c_top, 1)

    sentinel = jnp.int16(-32768)
    lo_masked = jnp.where(hi == h32.astype(jnp.int16), lo, sentinel)
    lo_thr, _ = _kth16(lo_masked, k2, done1)

    lo_u = (lo_thr + jnp.int32(32768)) & jnp.int32(0xFFFF)
    full = jnp.left_shift(h32, 16) | lo_u
    return jnp.where(done1 > 0, jnp.left_shift(h32, 16), full)


def _kl_body(p_ref, q_ref, o_ref):
    p = p_ref[...]
    q = q_ref[...]
    hp, lp = _halves(p)
    hq, lq = _halves(q)
    thr = _kth_largest_key(
        jnp.concatenate([hp, hq], axis=0),
        jnp.concatenate([lp, lq], axis=0),
        _K,
    )
    thrf = jax.lax.bitcast_convert_type(
        jnp.where(thr >= 0, thr, thr ^ jnp.int32(0x7FFFFFFF)), jnp.float32
    )
    mask = (p >= thrf[:_R]) | (q >= thrf[_R:])

    mp = jnp.max(p, axis=-1, keepdims=True)
    mq = jnp.max(q, axis=-1, keepdims=True)
    ep = jnp.where(mask, jnp.exp(p - mp), 0.0)
    eq = jnp.where(mask, jnp.exp(q - mq), 0.0)
    sp = jnp.sum(ep, axis=-1, keepdims=True)
    sq = jnp.sum(eq, axis=-1, keepdims=True)
    pn = ep * (1.0 / sp) + _EPS
    qn = eq * (1.0 / sq) + _EPS
    t = (pn - qn) * (jnp.log(pn) - jnp.log(qn))
    z = jnp.float32(1.0 + _V * _EPS)
    o_ref[...] = (0.5 / z) * jnp.sum(t, axis=-1, keepdims=True)


@jax.jit
def kernel(logits_p, logits_q):
    p = logits_p.reshape(_ROWS, _V)
    q = logits_q.reshape(_ROWS, _V)
    out = pl.pallas_call(
        _kl_body,
        grid=(_ROWS // _R,),
        in_specs=[
            pl.BlockSpec((_R, _V), lambda i: (i, 0)),
            pl.BlockSpec((_R, _V), lambda i: (i, 0)),
        ],
        out_specs=pl.BlockSpec((_R, 1), lambda i: (i, 0)),
        out_shape=jax.ShapeDtypeStruct((_ROWS, 1), jnp.float32),
    )(p, q)
    return out.reshape(logits_p.shape[0], logits_p.shape[1])


# speedup vs baseline: 1.1516x; 1.0240x over previous
import jax
import jax.numpy as jnp
from jax.experimental import pallas as pl

_EPS = 1e-8
_K = 64
_V = 32768
_R = 32
_ROWS = 512
_NCH = 64


def _keys(x):
    b = jax.lax.bitcast_convert_type(x, jnp.int32)
    return jnp.where(b >= 0, b, b ^ jnp.int32(0x7FFFFFFF))


def _count_ge16(data, cand):
    chw = _V // _NCH
    acc = jnp.zeros((data.shape[0], chw), jnp.int16)
    for c in range(_NCH):
        blk = jax.lax.slice_in_dim(data, c * chw, (c + 1) * chw, axis=1)
        acc = jnp.where(blk >= cand, acc + jnp.int16(1), acc)
    return jnp.sum(acc.astype(jnp.int32), axis=-1, keepdims=True)


def _kth16(data, k, done):
    cnt0 = _count_ge16(data, jnp.zeros((data.shape[0], 1), jnp.int16))
    prefix = jnp.where(cnt0 >= k, jnp.int32(0), jnp.int32(-32768))
    done = done | (cnt0 == k).astype(jnp.int32)

    def cond(state):
        i, _, done = state
        return jnp.logical_and(i < 15, jnp.logical_not(jnp.all(done > 0)))

    def body(state):
        i, prefix, done = state
        bit = jnp.left_shift(jnp.int32(1), 14 - i)
        cand = prefix | bit
        cnt = _count_ge16(data, cand.astype(jnp.int16))
        live_take = jnp.logical_and(cnt >= k, done == 0)
        prefix = jnp.where(live_take, cand, prefix)
        done = done | (cnt == k).astype(jnp.int32)
        return i + 1, prefix, done

    _, prefix, done = jax.lax.while_loop(cond, body, (jnp.int32(0), prefix, done))
    return prefix, done


def _halves(x):
    key = _keys(x)
    hi = jnp.right_shift(key, 16).astype(jnp.int16)
    lo = (key ^ jnp.int32(0x8000)).astype(jnp.int16)
    return hi, lo


def _kth_largest_key(hi, lo, k):
    rows = hi.shape[0]
    kvec = jnp.full((rows, 1), k, jnp.int32)
    h32, done1 = _kth16(hi, kvec, jnp.zeros((rows, 1), jnp.int32))

    c_top = jnp.where(
        h32 >= 32767,
        jnp.zeros_like(kvec),
        _count_ge16(hi, (h32 + 1).astype(jnp.int16)),
    )
    k2 = jnp.maximum(kvec - c_top, 1)

    sentinel = jnp.int16(-32768)
    lo_masked = jnp.where(hi == h32.astype(jnp.int16), lo, sentinel)
    lo_thr, _ = _kth16(lo_masked, k2, done1)

    lo_u = (lo_thr + jnp.int32(32768)) & jnp.int32(0xFFFF)
    full = jnp.left_shift(h32, 16) | lo_u
    return jnp.where(done1 > 0, jnp.left_shift(h32, 16), full)


def _kl_body(p_ref, q_ref, o_ref):
    p = p_ref[...]
    q = q_ref[...]
    hp, lp = _halves(p)
    hq, lq = _halves(q)
    thr = _kth_largest_key(
        jnp.concatenate([hp, hq], axis=0),
        jnp.concatenate([lp, lq], axis=0),
        _K,
    )
    thrf = jax.lax.bitcast_convert_type(
        jnp.where(thr >= 0, thr, thr ^ jnp.int32(0x7FFFFFFF)), jnp.float32
    )
    mask = (p >= thrf[:_R]) | (q >= thrf[_R:])

    mp = jnp.max(p, axis=-1, keepdims=True)
    mq = jnp.max(q, axis=-1, keepdims=True)
    ep = jnp.where(mask, jnp.exp(p - mp), 0.0)
    eq = jnp.where(mask, jnp.exp(q - mq), 0.0)
    sp = jnp.sum(ep, axis=-1, keepdims=True)
    sq = jnp.sum(eq, axis=-1, keepdims=True)
    pn = ep * (1.0 / sp) + _EPS
    qn = eq * (1.0 / sq) + _EPS
    t = (pn - qn) * (jnp.log(pn) - jnp.log(qn))
    z = jnp.float32(1.0 + _V * _EPS)
    o_ref[...] = (0.5 / z) * jnp.sum(t, axis=-1, keepdims=True)


@jax.jit
def kernel(logits_p, logits_q):
    p = logits_p.reshape(_ROWS, _V)
    q = logits_q.reshape(_ROWS, _V)
    out = pl.pallas_call(
        _kl_body,
        grid=(_ROWS // _R,),
        in_specs=[
            pl.BlockSpec((_R, _V), lambda i: (i, 0)),
            pl.BlockSpec((_R, _V), lambda i: (i, 0)),
        ],
        out_specs=pl.BlockSpec((_R, 1), lambda i: (i, 0)),
        out_shape=jax.ShapeDtypeStruct((_ROWS, 1), jnp.float32),
    )(p, q)
    return out.reshape(logits_p.shape[0], logits_p.shape[1])
